# baseline (speedup 1.0000x reference)
import functools
import numpy as np
import jax
import jax.numpy as jnp
from jax import lax
from jax.experimental import pallas as pl
from jax.experimental.pallas import tpu as pltpu
from jax.experimental.pallas import tpu_sc as plsc

N = 10000
E = 320000
D = 128
H = 8
DK = 16
L = 2
R = 50
FF = 4 * D
W = D + 16

NC = 2
NS = 16
NW = NC * NS
EPW = E // NW
C = 40
NCHUNK = EPW // C
RPT = 624
REM = N - NS * RPT


_BLK = 1000


def _qkv_body(x_ref, w_ref, bq_ref, q_ref, k_ref, v_ref):
    x = x_ref[...]
    q_ref[...] = jnp.dot(x, w_ref[0], preferred_element_type=jnp.float32) + bq_ref[...]
    k_ref[...] = jnp.dot(x, w_ref[1], preferred_element_type=jnp.float32)
    v_ref[...] = jnp.dot(x, w_ref[2], preferred_element_type=jnp.float32)


def _tc_qkv(x, wstack, bq):
    return pl.pallas_call(
        _qkv_body,
        grid=(N // _BLK,),
        in_specs=[
            pl.BlockSpec((_BLK, D), lambda i: (i, 0)),
            pl.BlockSpec((3, D, D), lambda i: (0, 0, 0)),
            pl.BlockSpec((1, D), lambda i: (0, 0)),
        ],
        out_specs=[
            pl.BlockSpec((_BLK, D), lambda i: (i, 0)),
            pl.BlockSpec((_BLK, D), lambda i: (i, 0)),
            pl.BlockSpec((_BLK, D), lambda i: (i, 0)),
        ],
        out_shape=[
            jax.ShapeDtypeStruct((N, D), jnp.float32),
            jax.ShapeDtypeStruct((N, D), jnp.float32),
            jax.ShapeDtypeStruct((N, D), jnp.float32),
        ],
    )(x, wstack, bq)


def _sc_edge_body(q_hbm, k_hbm, v_hbm, rel_hbm, pidx_hbm, zacc_hbm,
                  out_hbm,
                  acc,
                  pidx0, pidx1,
                  kr0, kr1, qr0, qr1, vr0, vr1, re0, re1,
                  msg, sem0, sem1):
    c = lax.axis_index("c")
    s = lax.axis_index("s")
    wid = c * NS + s

    bufs = ((pidx0, kr0, qr0, vr0, re0, sem0),
            (pidx1, kr1, qr1, vr1, re1, sem1))

    zero16 = jnp.zeros((16,), jnp.float32)

    row0 = s * RPT
    pltpu.sync_copy(zacc_hbm.at[pl.ds(row0, RPT)], acc.at[pl.ds(row0, RPT)])

    @pl.when(s == NS - 1)
    def _zero_tail():
        pltpu.sync_copy(zacc_hbm.at[pl.ds(NS * RPT, REM)],
                        acc.at[pl.ds(NS * RPT, REM)])

    plsc.subcore_barrier()

    iota = lax.iota(jnp.int32, 16)
    inv_scale = 1.0 / float(np.sqrt(DK))
    shuf = [(iota + sh) & 15 for sh in (8, 4, 2, 1)]

    dnums = lax.GatherDimensionNumbers(
        offset_dims=(), collapsed_slice_dims=(0,), start_index_map=(0,))

    def _allsum(t):
        for sx in shuf:
            t = t + lax.gather(t, sx[:, None], dnums, slice_sizes=(1,),
                               mode=lax.GatherScatterMode.PROMISE_IN_BOUNDS)
        return t

    def _load_idx(g, buf):
        pltpu.sync_copy(pidx_hbm.at[wid * NCHUNK + g], buf[0])

    def _gather_copies(buf):
        pidx, kr, qr, vr, re, sem = buf
        return (
            pltpu.make_async_copy(k_hbm.at[pidx.at[0]], kr, sem),
            pltpu.make_async_copy(q_hbm.at[pidx.at[1]], qr, sem),
            pltpu.make_async_copy(v_hbm.at[pidx.at[0]], vr, sem),
            pltpu.make_async_copy(rel_hbm.at[pidx.at[2]], re, sem),
        )

    def _start_gathers(buf):
        for cp in _gather_copies(buf):
            cp.start()

    def _wait_gathers(buf):
        for cp in _gather_copies(buf):
            cp.wait()

    def _compute_chunk(buf):
        pidx, kr, qr, vr, re, sem = buf

        def _edge(i, ecarry):
            rel = re[i, :]
            zv = zero16
            for h in range(H):
                sl = pl.ds(h * 16, 16)
                kh = kr[i, sl]
                qh = qr[i, sl]
                vh = vr[i, sl]
                sb = _allsum((kh + rel) * qh) * inv_scale
                se = jnp.exp(jnp.clip(sb, -10.0, 10.0))
                msg[i, sl] = (vh + rel) * se
                zv = jnp.where(iota == h, se, zv)
            msg[i, pl.ds(D, 16)] = zv
            return ecarry

        lax.fori_loop(0, C, _edge, 0)
        pltpu.sync_copy(msg, acc.at[pidx.at[1]], add=True)

    _load_idx(0, bufs[0])
    _start_gathers(bufs[0])

    def _step(t2, carry):
        for b in (0, 1):
            g = t2 * 2 + b
            cur = bufs[b]
            nxt = bufs[1 - b]

            @pl.when(g < NCHUNK - 1)
            def _prefetch():
                _load_idx(g + 1, nxt)
                _start_gathers(nxt)

            _wait_gathers(cur)
            _compute_chunk(cur)
        return carry

    lax.fori_loop(0, NCHUNK // 2, _step, 0)
    plsc.subcore_barrier()

    pltpu.sync_copy(acc.at[pl.ds(row0, RPT)], out_hbm.at[c, pl.ds(row0, RPT)])

    @pl.when(s == NS - 1)
    def _copy_tail():
        pltpu.sync_copy(acc.at[pl.ds(NS * RPT, REM)],
                        out_hbm.at[c, pl.ds(NS * RPT, REM)])


_sc_edge = functools.partial(
    pl.kernel,
    _sc_edge_body,
    out_type=jax.ShapeDtypeStruct((NC, N, W), jnp.float32),
    mesh=plsc.VectorSubcoreMesh(core_axis_name="c", subcore_axis_name="s"),
    compiler_params=pltpu.CompilerParams(use_tc_tiling_on_sc=False),
    scratch_types=(
        [
            pltpu.VMEM_SHARED((N, W), jnp.float32),
            pltpu.VMEM((3, C), jnp.int32),
            pltpu.VMEM((3, C), jnp.int32),
        ]
        + [pltpu.VMEM((C, D), jnp.float32)] * 6
        + [pltpu.VMEM((C, 16), jnp.float32)] * 2
        + [
            pltpu.VMEM((C, W), jnp.float32),
            pltpu.SemaphoreType.DMA,
            pltpu.SemaphoreType.DMA,
        ]
    ),
)()


def _ln(a, g, b, eps=1e-5):
    m = jnp.mean(a, axis=-1, keepdims=True)
    v = jnp.mean((a - m) ** 2, axis=-1, keepdims=True)
    return g * (a - m) / jnp.sqrt(v + eps) + b


def _post_body(acc_ref, x_ref, e2_ref, wo_ref, bo_ref, g1_ref, b1n_ref,
               w1_ref, b1_ref, w2_ref, b2_ref, g2_ref, b2n_ref, out_ref):
    a0 = acc_ref[0]
    a1 = acc_ref[1]
    wv = a0[:, :D] + a1[:, :D]
    zh = a0[:, D:] + a1[:, D:]
    z128 = jnp.dot(zh, e2_ref[...], preferred_element_type=jnp.float32)
    o = wv / (z128 + 1e-12)
    a = x_ref[...] + jnp.dot(o, wo_ref[...], preferred_element_type=jnp.float32) + bo_ref[...]
    x1 = _ln(a, g1_ref[...], b1n_ref[...])
    hmid = jnp.maximum(jnp.dot(x1, w1_ref[...], preferred_element_type=jnp.float32) + b1_ref[...], 0.0)
    hout = jnp.dot(hmid, w2_ref[...], preferred_element_type=jnp.float32) + b2_ref[...]
    out_ref[...] = _ln(x1 + hout, g2_ref[...], b2n_ref[...])


def _tc_post(acc, x, e2, wo, bo, g1, b1n, w1, b1, w2, b2, g2, b2n):
    return pl.pallas_call(
        _post_body,
        grid=(N // _BLK,),
        in_specs=[
            pl.BlockSpec((NC, _BLK, W), lambda i: (0, i, 0)),
            pl.BlockSpec((_BLK, D), lambda i: (i, 0)),
            pl.BlockSpec((16, D), lambda i: (0, 0)),
            pl.BlockSpec((D, D), lambda i: (0, 0)),
            pl.BlockSpec((1, D), lambda i: (0, 0)),
            pl.BlockSpec((1, D), lambda i: (0, 0)),
            pl.BlockSpec((1, D), lambda i: (0, 0)),
            pl.BlockSpec((D, FF), lambda i: (0, 0)),
            pl.BlockSpec((1, FF), lambda i: (0, 0)),
            pl.BlockSpec((FF, D), lambda i: (0, 0)),
            pl.BlockSpec((1, D), lambda i: (0, 0)),
            pl.BlockSpec((1, D), lambda i: (0, 0)),
            pl.BlockSpec((1, D), lambda i: (0, 0)),
        ],
        out_specs=pl.BlockSpec((_BLK, D), lambda i: (i, 0)),
        out_shape=jax.ShapeDtypeStruct((N, D), jnp.float32),
    )(acc, x, e2, wo, bo, g1, b1n, w1, b1, w2, b2, g2, b2n)


_E2 = np.zeros((16, D), np.float32)
for _h in range(H):
    _E2[_h, _h * DK:(_h + 1) * DK] = 1.0


def kernel(x, edge_index, edges, rel_embed, Wq, bq, Wk, Wv, Wo, bo,
           ln1_g, ln1_b, W1, b1, W2, b2, ln2_g, ln2_b):
    src = edge_index[0]
    dst = edge_index[1]
    e2 = jnp.asarray(_E2)
    zacc = jnp.zeros((N, W), jnp.float32)
    pidx = jnp.stack(
        [src.reshape(E // C, C), dst.reshape(E // C, C),
         edges.reshape(E // C, C)], axis=1)
    for i in range(L):
        wstack = jnp.stack([Wq[i], Wk[i], Wv[i]])
        q, k, v = _tc_qkv(x, wstack, bq[i].reshape(1, D))
        acc = _sc_edge(q, k, v, rel_embed, pidx, zacc)
        x = _tc_post(acc, x, e2,
                     Wo[i], bo[i].reshape(1, D),
                     ln1_g[i].reshape(1, D), ln1_b[i].reshape(1, D),
                     W1[i], b1[i].reshape(1, FF),
                     W2[i], b2[i].reshape(1, D),
                     ln2_g[i].reshape(1, D), ln2_b[i].reshape(1, D))
    return x
